# baseline (speedup 1.0000x reference)
import functools

import jax
import jax.numpy as jnp
from jax import lax
from jax.experimental import pallas as pl
from jax.experimental.pallas import tpu as pltpu
from jax.experimental.pallas import tpu_sc as plsc

BATCH = 16384
EMB_DIM = 64
NUM_NODES = 1_000_000

_info = plsc.get_sparse_core_info()
_NC, _NS = _info.num_cores, _info.num_subcores

_MAIN_END = 999936
_GROUP = 256
_GPC = (_MAIN_END // _GROUP) // 2
_MAXLG = (_GPC + 15) // 16
_BKT_CAP = BATCH + _MAXLG * 16 + 16
_SLOT = 128
_BSENT = 16384


@functools.partial(
    pl.kernel,
    mesh=plsc.VectorSubcoreMesh(core_axis_name="c", subcore_axis_name="s"),
    out_type=(
        jax.ShapeDtypeStruct((BATCH + 1, _SLOT), jnp.float32),
        jax.ShapeDtypeStruct((16, _SLOT), jnp.float32),
    ),
    scratch_types=[
        pltpu.VMEM((512,), jnp.int32),
        pltpu.VMEM((_BKT_CAP,), jnp.int32),
        pltpu.VMEM((256,), jnp.int32),
        pltpu.VMEM((256,), jnp.int32),
        pltpu.VMEM((256,), jnp.int32),
        pltpu.VMEM((2, EMB_DIM, _GROUP), jnp.float32),
        pltpu.VMEM((32, _SLOT), jnp.float32),
        pltpu.VMEM((EMB_DIM, 64), jnp.float32),
        pltpu.SemaphoreType.DMA,
        pltpu.SemaphoreType.DMA,
    ],
    compiler_params=pltpu.CompilerParams(needs_layout_passes=False),
)
def _gather_kernel(ids_hbm, tab_t_hbm, tail_hbm, out_hbm, dummy_hbm,
                   win_v, bkt_v, pcnt_v, offs_v, curs_v, buf_v, ring_v,
                   tail_v, sem_g, sem_r):
    core = lax.axis_index("c")
    tec = lax.axis_index("s")
    nlg = (_GPC - tec + 15) // 16

    iota16 = lax.broadcasted_iota(jnp.int32, (16,), 0)
    zeros16 = jnp.zeros((16,), jnp.int32)

    def scal(ref, i):
        v = plsc.load_gather(ref, [jnp.full((16,), i, jnp.int32)])
        return v[0]

    def masks(c, j):
        idv = win_v[pl.ds(j * 16, 16)]
        bv = jnp.full((16,), c * 512 + j * 16, jnp.int32) + iota16
        gl = (idv >> 8) - core * _GPC
        m = ((idv < _MAIN_END)
             & (gl >= 0) & (gl < _GPC)
             & ((gl & 15) == tec))
        lg = lax.max(lax.min(gl >> 4, jnp.full((16,), 255, jnp.int32)),
                     zeros16)
        return idv, bv, m, lg

    def scan_ids(body16):
        def win_body(c, carry):
            pltpu.sync_copy(ids_hbm.at[pl.ds(c * 512, 512)], win_v)

            def vec_body(j, carry2):
                body16(c, j)
                return carry2

            return lax.fori_loop(0, 32, vec_body, carry)

        lax.fori_loop(0, BATCH // 512, win_body, 0)

    for k in range(16):
        curs_v[pl.ds(16 * k, 16)] = zeros16

    def count16(c, j):
        _, _, m, lg = masks(c, j)
        ordv, lastm = plsc.scan_count(lg, m)
        cur = plsc.load_gather(curs_v, [lg])
        plsc.store_scatter(curs_v, [lg], cur + ordv, mask=m & lastm)

    scan_ids(count16)

    running = jnp.int32(0)
    for k in range(16):
        c = curs_v[pl.ds(16 * k, 16)]
        p = (c + 15) & ~15
        pcnt_v[pl.ds(16 * k, 16)] = p
        s = plsc.cumsum(p)
        offs_v[pl.ds(16 * k, 16)] = s - p + running
        running = running + s[15]
    total_entries = running

    sent16 = jnp.full((16,), _BSENT, jnp.int32)

    def fill_body(i, carry):
        bkt_v[pl.ds(i * 16, 16)] = sent16
        return carry

    lax.fori_loop(0, (total_entries + 15) // 16, fill_body, 0)
    for k in range(16):
        curs_v[pl.ds(16 * k, 16)] = offs_v[pl.ds(16 * k, 16)]

    def place16(c, j):
        idv, bv, m, lg = masks(c, j)
        ordv, lastm = plsc.scan_count(lg, m)
        cur = plsc.load_gather(curs_v, [lg])
        slot = lax.min(cur + ordv - 1, jnp.full((16,), _BKT_CAP - 1, jnp.int32))
        slot = lax.max(slot, zeros16)
        val = ((idv & (_GROUP - 1)) << 15) | bv
        plsc.store_scatter(bkt_v, [slot], val, mask=m)
        plsc.store_scatter(curs_v, [lg], cur + ordv, mask=m & lastm)

    scan_ids(place16)

    def fetch(lg, p):
        n0 = (core * _GPC + lg * 16 + tec) * _GROUP
        n0 = pl.multiple_of(n0, 128)
        pltpu.async_copy(tab_t_hbm.at[:, pl.ds(n0, _GROUP)], buf_v.at[p], sem_g)

    @pl.when(nlg > 0)
    def _():
        fetch(0, 0)

    rows16 = [jnp.full((16,), 16 * d, jnp.int32) + iota16 for d in range(4)]

    def drain_chunk():
        pltpu.make_async_copy(
            dummy_hbm, ring_v.at[pl.ds(0, 16)], sem_r
        ).wait()

    def chunk_work(c, gchunk, src_ref, lane_cap):
        val = bkt_v[pl.ds(c * 16, 16)]
        bv = val & 32767
        lanes_v = lax.min(val >> 15, jnp.full((16,), lane_cap, jnp.int32))
        slot0 = (gchunk % 2) * 16

        @pl.when(gchunk >= 2)
        def _():
            drain_chunk()

        for l in range(16):
            lane = jnp.full((16,), lanes_v[l], jnp.int32)
            for d in range(4):
                vals = plsc.load_gather(src_ref, [rows16[d], lane])
                ring_v[slot0 + l, pl.ds(16 * d, 16)] = vals
            row = lax.min(bv[l], jnp.int32(BATCH))
            pltpu.async_copy(ring_v.at[slot0 + l], out_hbm.at[row], sem_r)
        return gchunk + 1

    def group_body(lg, gchunk):
        p = lg % 2
        pltpu.make_async_copy(
            tab_t_hbm.at[:, pl.ds(0, _GROUP)], buf_v.at[p], sem_g
        ).wait()

        @pl.when(lg + 1 < nlg)
        def _():
            fetch(lg + 1, 1 - p)

        cstart = scal(offs_v, lg) // 16
        nck = scal(pcnt_v, lg) // 16

        def inner(k, gc):
            return chunk_work(cstart + k, gc, buf_v.at[p], _GROUP - 1)

        return lax.fori_loop(0, nck, inner, gchunk)

    gchunk = lax.fori_loop(0, nlg, group_body, jnp.int32(0))

    @pl.when((core == 1) & (tec == 15))
    def _():
        pltpu.sync_copy(tail_hbm, tail_v)

    def tail_scan(c, carry):
        pltpu.sync_copy(ids_hbm.at[pl.ds(c * 512, 512)], win_v)

        def tail_vec(j, gc):
            idv = win_v[pl.ds(j * 16, 16)]
            bv_full = jnp.full((16,), c * 512 + j * 16, jnp.int32) + iota16
            m = idv >= _MAIN_END
            npop = plsc.all_reduce_population_count(m)

            def do_tail(gc_in):
                val = ((idv & (_GROUP - 1)) << 15) | bv_full
                bkt_v[pl.ds(_BKT_CAP - 16, 16)] = sent16
                plsc.store_compressed(
                    bkt_v.at[pl.ds(_BKT_CAP - 16, 16)], val, mask=m)
                return chunk_work(_BKT_CAP // 16 - 1, gc_in, tail_v, 63)

            return lax.cond(npop[0] > 0, do_tail, lambda x: x, gc)

        return lax.fori_loop(0, 32, tail_vec, carry)

    gchunk = lax.cond(
        (core == 1) & (tec == 15),
        lambda gc: lax.fori_loop(0, BATCH // 512, tail_scan, gc),
        lambda gc: gc,
        gchunk,
    )

    for k in range(2):
        @pl.when(gchunk > k)
        def _():
            drain_chunk()


def kernel(ids, table):
    ids32 = ids.astype(jnp.int32)
    tab_t = table.T
    tail = table[_MAIN_END:].T
    out, _ = _gather_kernel(ids32, tab_t, tail)
    return out[:BATCH, :EMB_DIM]


# speedup vs baseline: 6.1598x; 6.1598x over previous
import functools

import jax
import jax.numpy as jnp
from jax import lax
from jax.experimental import pallas as pl
from jax.experimental.pallas import tpu as pltpu
from jax.experimental.pallas import tpu_sc as plsc

BATCH = 16384
EMB_DIM = 64
NUM_NODES = 1_000_000

_info = plsc.get_sparse_core_info()
_NC, _NS = _info.num_cores, _info.num_subcores

_MAIN_END = 999936
_GROUP = 256
_GPC = (_MAIN_END // _GROUP) // 2
_MAXLG = (_GPC + 15) // 16
_BKT_CAP = BATCH + _MAXLG * 16 + 16
_SLOT = 128
_BSENT = 16384


@functools.partial(
    pl.kernel,
    mesh=plsc.VectorSubcoreMesh(core_axis_name="c", subcore_axis_name="s"),
    out_type=(
        jax.ShapeDtypeStruct((BATCH + 32, _SLOT), jnp.float32),
        jax.ShapeDtypeStruct((16, _SLOT), jnp.float32),
    ),
    scratch_types=[
        pltpu.VMEM((512,), jnp.int32),
        pltpu.VMEM((_BKT_CAP,), jnp.int32),
        pltpu.VMEM((256,), jnp.int32),
        pltpu.VMEM((256,), jnp.int32),
        pltpu.VMEM((256,), jnp.int32),
        pltpu.VMEM((2, EMB_DIM, _GROUP), jnp.float32),
        pltpu.VMEM((64, _SLOT), jnp.float32),
        pltpu.VMEM((EMB_DIM, 64), jnp.float32),
        pltpu.SemaphoreType.DMA,
        pltpu.SemaphoreType.DMA,
    ],
    compiler_params=pltpu.CompilerParams(needs_layout_passes=False),
)
def _gather_kernel(ids_hbm, tab_t_hbm, tail_hbm, out_hbm, dummy_hbm,
                   win_v, bkt_v, pcnt_v, offs_v, curs_v, buf_v, ring_v,
                   tail_v, sem_g, sem_r):
    core = lax.axis_index("c")
    tec = lax.axis_index("s")
    nlg = (_GPC - tec + 15) // 16

    iota16 = lax.broadcasted_iota(jnp.int32, (16,), 0)
    zeros16 = jnp.zeros((16,), jnp.int32)

    def scal(ref, i):
        v = plsc.load_gather(ref, [jnp.full((16,), i, jnp.int32)])
        return v[0]

    def masks(c, j):
        idv = win_v[pl.ds(j * 16, 16)]
        bv = jnp.full((16,), c * 512 + j * 16, jnp.int32) + iota16
        gl = (idv >> 8) - core * _GPC
        m = ((idv < _MAIN_END)
             & (gl >= 0) & (gl < _GPC)
             & ((gl & 15) == tec))
        lg = lax.max(lax.min(gl >> 4, jnp.full((16,), 255, jnp.int32)),
                     zeros16)
        return idv, bv, m, lg

    def scan_ids(body16):
        def win_body(c, carry):
            pltpu.sync_copy(ids_hbm.at[pl.ds(c * 512, 512)], win_v)

            def vec_body(j, carry2):
                body16(c, j)
                return carry2

            return lax.fori_loop(0, 32, vec_body, carry)

        lax.fori_loop(0, BATCH // 512, win_body, 0)

    for k in range(16):
        curs_v[pl.ds(16 * k, 16)] = zeros16

    def count16(c, j):
        _, _, m, lg = masks(c, j)
        ordv, lastm = plsc.scan_count(lg, m)
        cur = plsc.load_gather(curs_v, [lg])
        plsc.store_scatter(curs_v, [lg], cur + ordv, mask=m & lastm)

    scan_ids(count16)

    running = jnp.int32(0)
    for k in range(16):
        c = curs_v[pl.ds(16 * k, 16)]
        p = (c + 15) & ~15
        pcnt_v[pl.ds(16 * k, 16)] = p
        s = plsc.cumsum(p)
        offs_v[pl.ds(16 * k, 16)] = s - p + running
        running = running + s[15]
    total_entries = running

    sent16 = jnp.full((16,), _BSENT, jnp.int32)

    def fill_body(i, carry):
        bkt_v[pl.ds(i * 16, 16)] = sent16
        return carry

    lax.fori_loop(0, (total_entries + 15) // 16, fill_body, 0)
    for k in range(16):
        curs_v[pl.ds(16 * k, 16)] = offs_v[pl.ds(16 * k, 16)]

    def place16(c, j):
        idv, bv, m, lg = masks(c, j)
        ordv, lastm = plsc.scan_count(lg, m)
        cur = plsc.load_gather(curs_v, [lg])
        slot = lax.min(cur + ordv - 1, jnp.full((16,), _BKT_CAP - 1, jnp.int32))
        slot = lax.max(slot, zeros16)
        val = ((idv & (_GROUP - 1)) << 15) | bv
        plsc.store_scatter(bkt_v, [slot], val, mask=m)
        plsc.store_scatter(curs_v, [lg], cur + ordv, mask=m & lastm)

    scan_ids(place16)

    def fetch(lg, p):
        n0 = (core * _GPC + lg * 16 + tec) * _GROUP
        n0 = pl.multiple_of(n0, 128)
        pltpu.async_copy(tab_t_hbm.at[:, pl.ds(n0, _GROUP)], buf_v.at[p], sem_g)

    @pl.when(nlg > 0)
    def _():
        fetch(0, 0)

    rows16 = [jnp.full((16,), 16 * d, jnp.int32) + iota16 for d in range(4)]

    def drain_chunk():
        pltpu.make_async_copy(
            dummy_hbm, ring_v.at[pl.ds(0, 16)], sem_r
        ).wait()

    def chunk_work(c, gchunk, src_ref, lane_cap):
        val = bkt_v[pl.ds(c * 16, 16)]
        bv = val & 32767
        lanes_v = lax.min(val >> 15, jnp.full((16,), lane_cap, jnp.int32))
        slot0 = (gchunk % 4) * 16

        @pl.when(gchunk >= 4)
        def _():
            drain_chunk()

        for l in range(16):
            lane = jnp.full((16,), lanes_v[l], jnp.int32)
            for d in range(4):
                vals = plsc.load_gather(src_ref, [rows16[d], lane])
                ring_v[slot0 + l, pl.ds(16 * d, 16)] = vals
            row = lax.min(bv[l], jnp.int32(BATCH))
            row = row + (row >> 14) * (core * 16 + tec)
            pltpu.async_copy(ring_v.at[slot0 + l], out_hbm.at[row], sem_r)
        return gchunk + 1

    def group_body(lg, gchunk):
        p = lg % 2
        pltpu.make_async_copy(
            tab_t_hbm.at[:, pl.ds(0, _GROUP)], buf_v.at[p], sem_g
        ).wait()

        @pl.when(lg + 1 < nlg)
        def _():
            fetch(lg + 1, 1 - p)

        cstart = scal(offs_v, lg) // 16
        nck = scal(pcnt_v, lg) // 16

        def inner(k, gc):
            return chunk_work(cstart + k, gc, buf_v.at[p], _GROUP - 1)

        return lax.fori_loop(0, nck, inner, gchunk)

    gchunk = lax.fori_loop(0, nlg, group_body, jnp.int32(0))

    @pl.when((core == 1) & (tec == 15))
    def _():
        pltpu.sync_copy(tail_hbm, tail_v)

    def tail_scan(c, carry):
        pltpu.sync_copy(ids_hbm.at[pl.ds(c * 512, 512)], win_v)

        def tail_vec(j, gc):
            idv = win_v[pl.ds(j * 16, 16)]
            bv_full = jnp.full((16,), c * 512 + j * 16, jnp.int32) + iota16
            m = idv >= _MAIN_END
            npop = plsc.all_reduce_population_count(m)

            def do_tail(gc_in):
                val = ((idv & (_GROUP - 1)) << 15) | bv_full
                bkt_v[pl.ds(_BKT_CAP - 16, 16)] = sent16
                plsc.store_compressed(
                    bkt_v.at[pl.ds(_BKT_CAP - 16, 16)], val, mask=m)
                return chunk_work(_BKT_CAP // 16 - 1, gc_in, tail_v, 63)

            return lax.cond(npop[0] > 0, do_tail, lambda x: x, gc)

        return lax.fori_loop(0, 32, tail_vec, carry)

    gchunk = lax.cond(
        (core == 1) & (tec == 15),
        lambda gc: lax.fori_loop(0, BATCH // 512, tail_scan, gc),
        lambda gc: gc,
        gchunk,
    )

    for k in range(4):
        @pl.when(gchunk > k)
        def _():
            drain_chunk()


def kernel(ids, table):
    ids32 = ids.astype(jnp.int32)
    tab_t = table.T
    tail = table[_MAIN_END:].T
    out, _ = _gather_kernel(ids32, tab_t, tail)
    return out[:BATCH, :EMB_DIM]


# speedup vs baseline: 7.4269x; 1.2057x over previous
import functools

import jax
import jax.numpy as jnp
from jax import lax
from jax.experimental import pallas as pl
from jax.experimental.pallas import tpu as pltpu
from jax.experimental.pallas import tpu_sc as plsc

BATCH = 16384
EMB_DIM = 64
NUM_NODES = 1_000_000

_info = plsc.get_sparse_core_info()
_NC, _NS = _info.num_cores, _info.num_subcores

_MAIN_END = 999936
_GROUP = 256
_GPC = (_MAIN_END // _GROUP) // 2
_MAXLG = (_GPC + 15) // 16
_BKT_CAP = BATCH + _MAXLG * 16 + 16
_SLOT = 128


@functools.partial(
    pl.kernel,
    mesh=plsc.VectorSubcoreMesh(core_axis_name="c", subcore_axis_name="s"),
    out_type=(
        jax.ShapeDtypeStruct((BATCH, _SLOT), jnp.float32),
        jax.ShapeDtypeStruct((16, _SLOT), jnp.float32),
    ),
    scratch_types=[
        pltpu.VMEM((512,), jnp.int32),
        pltpu.VMEM((_BKT_CAP,), jnp.int32),
        pltpu.VMEM((256,), jnp.int32),
        pltpu.VMEM((256,), jnp.int32),
        pltpu.VMEM((2, EMB_DIM, _GROUP), jnp.float32),
        pltpu.VMEM((64, _SLOT), jnp.float32),
        pltpu.VMEM((EMB_DIM, 64), jnp.float32),
        pltpu.SemaphoreType.DMA,
        pltpu.SemaphoreType.DMA,
    ],
    compiler_params=pltpu.CompilerParams(needs_layout_passes=False),
)
def _gather_kernel(ids_hbm, tab_t_hbm, tail_hbm, out_hbm, dummy_hbm,
                   win_v, bkt_v, offs_v, curs_v, buf_v, ring_v,
                   tail_v, sem_g, sem_r):
    core = lax.axis_index("c")
    tec = lax.axis_index("s")
    nlg = (_GPC - tec + 15) // 16

    iota16 = lax.broadcasted_iota(jnp.int32, (16,), 0)
    zeros16 = jnp.zeros((16,), jnp.int32)

    def scal(ref, i):
        v = plsc.load_gather(ref, [jnp.full((16,), i, jnp.int32)])
        return v[0]

    def masks(c, j):
        idv = win_v[pl.ds(j * 16, 16)]
        bv = jnp.full((16,), c * 512 + j * 16, jnp.int32) + iota16
        gl = (idv >> 8) - core * _GPC
        m = ((idv < _MAIN_END)
             & (gl >= 0) & (gl < _GPC)
             & ((gl & 15) == tec))
        lg = lax.max(lax.min(gl >> 4, jnp.full((16,), 255, jnp.int32)),
                     zeros16)
        return idv, bv, m, lg

    def scan_ids(body16):
        def win_body(c, carry):
            pltpu.sync_copy(ids_hbm.at[pl.ds(c * 512, 512)], win_v)

            def vec_body(j, carry2):
                body16(c, j)
                return carry2

            return lax.fori_loop(0, 32, vec_body, carry)

        lax.fori_loop(0, BATCH // 512, win_body, 0)

    for k in range(16):
        curs_v[pl.ds(16 * k, 16)] = zeros16

    def count16(c, j):
        _, _, m, lg = masks(c, j)
        ordv, lastm = plsc.scan_count(lg, m)
        cur = plsc.load_gather(curs_v, [lg])
        plsc.store_scatter(curs_v, [lg], cur + ordv, mask=m & lastm)

    scan_ids(count16)

    running = jnp.int32(0)
    for k in range(16):
        c = curs_v[pl.ds(16 * k, 16)]
        p = (c + 15) & ~15
        s = plsc.cumsum(p)
        offs_v[pl.ds(16 * k, 16)] = s - p + running
        running = running + s[15]
    for k in range(16):
        curs_v[pl.ds(16 * k, 16)] = offs_v[pl.ds(16 * k, 16)]

    def place16(c, j):
        idv, bv, m, lg = masks(c, j)
        ordv, lastm = plsc.scan_count(lg, m)
        cur = plsc.load_gather(curs_v, [lg])
        slot = lax.min(cur + ordv - 1, jnp.full((16,), _BKT_CAP - 1, jnp.int32))
        slot = lax.max(slot, zeros16)
        val = ((idv & (_GROUP - 1)) << 15) | bv
        plsc.store_scatter(bkt_v, [slot], val, mask=m)
        plsc.store_scatter(curs_v, [lg], cur + ordv, mask=m & lastm)

    scan_ids(place16)

    def fetch(lg, p):
        n0 = (core * _GPC + lg * 16 + tec) * _GROUP
        n0 = pl.multiple_of(n0, 128)
        pltpu.async_copy(tab_t_hbm.at[:, pl.ds(n0, _GROUP)], buf_v.at[p], sem_g)

    @pl.when(nlg > 0)
    def _():
        fetch(0, 0)

    rows16 = [jnp.full((16,), 16 * d, jnp.int32) + iota16 for d in range(4)]

    def drain_one():
        pltpu.make_async_copy(dummy_hbm.at[0], ring_v.at[0], sem_r).wait()

    def chunk_work(c, issued, drained, src_ref, lane_cap, nreal):
        val = bkt_v[pl.ds(c * 16, 16)]
        bv = val & 32767
        lanes_v = lax.min(val >> 15, jnp.full((16,), lane_cap, jnp.int32))

        dr = lax.max(issued - drained - jnp.int32(48), jnp.int32(0))

        def dbody(i, cc):
            drain_one()
            return cc

        lax.fori_loop(0, dr, dbody, 0)
        drained = drained + dr

        for l in range(16):
            @pl.when(l < nreal)
            def _():
                lane = jnp.full((16,), lanes_v[l], jnp.int32)
                slot = (issued + l) & 63
                for d in range(4):
                    vals = plsc.load_gather(src_ref, [rows16[d], lane])
                    ring_v[slot, pl.ds(16 * d, 16)] = vals
                pltpu.async_copy(ring_v.at[slot], out_hbm.at[bv[l]], sem_r)
        return issued + nreal, drained

    def group_body(lg, carry):
        issued, drained = carry
        p = lg % 2
        pltpu.make_async_copy(
            tab_t_hbm.at[:, pl.ds(0, _GROUP)], buf_v.at[p], sem_g
        ).wait()

        @pl.when(lg + 1 < nlg)
        def _():
            fetch(lg + 1, 1 - p)

        off = scal(offs_v, lg)
        cstart = off // 16
        real = scal(curs_v, lg) - off
        nck = (real + 15) // 16

        def inner(k, cc):
            iss, drn = cc
            nreal = lax.min(real - 16 * k, jnp.int32(16))
            return chunk_work(cstart + k, iss, drn, buf_v.at[p],
                              _GROUP - 1, nreal)

        return lax.fori_loop(0, nck, inner, (issued, drained))

    issued, drained = lax.fori_loop(
        0, nlg, group_body, (jnp.int32(0), jnp.int32(0)))

    @pl.when((core == 1) & (tec == 15))
    def _():
        pltpu.sync_copy(tail_hbm, tail_v)

    def tail_scan(c, carry):
        pltpu.sync_copy(ids_hbm.at[pl.ds(c * 512, 512)], win_v)

        def tail_vec(j, cc):
            idv = win_v[pl.ds(j * 16, 16)]
            bv_full = jnp.full((16,), c * 512 + j * 16, jnp.int32) + iota16
            m = idv >= _MAIN_END
            npop = plsc.all_reduce_population_count(m)

            def do_tail(cc_in):
                iss, drn = cc_in
                val = ((idv & (_GROUP - 1)) << 15) | bv_full
                plsc.store_compressed(
                    bkt_v.at[pl.ds(_BKT_CAP - 16, 16)], val, mask=m)
                return chunk_work(_BKT_CAP // 16 - 1, iss, drn, tail_v,
                                  63, npop[0])

            return lax.cond(npop[0] > 0, do_tail, lambda x: x, cc)

        return lax.fori_loop(0, 32, tail_vec, carry)

    issued, drained = lax.cond(
        (core == 1) & (tec == 15),
        lambda cc: lax.fori_loop(0, BATCH // 512, tail_scan, cc),
        lambda cc: cc,
        (issued, drained),
    )

    def final_drain(i, cc):
        drain_one()
        return cc

    lax.fori_loop(0, issued - drained, final_drain, 0)


def kernel(ids, table):
    ids32 = ids.astype(jnp.int32)
    tab_t = table.T
    tail = table[_MAIN_END:].T
    out, _ = _gather_kernel(ids32, tab_t, tail)
    return out[:, :EMB_DIM]


# speedup vs baseline: 8.7670x; 1.1804x over previous
import functools

import jax
import jax.numpy as jnp
from jax import lax
from jax.experimental import pallas as pl
from jax.experimental.pallas import tpu as pltpu
from jax.experimental.pallas import tpu_sc as plsc

BATCH = 16384
EMB_DIM = 64
NUM_NODES = 1_000_000

_info = plsc.get_sparse_core_info()
_NC, _NS = _info.num_cores, _info.num_subcores

_MAIN_END = 999936
_GROUP = 256
_GPC = (_MAIN_END // _GROUP) // 2
_MAXLG = (_GPC + 15) // 16
_BKT_CAP = BATCH + _MAXLG * 16 + 16
_SLOT = 128


@functools.partial(
    pl.kernel,
    mesh=plsc.VectorSubcoreMesh(core_axis_name="c", subcore_axis_name="s"),
    out_type=(
        jax.ShapeDtypeStruct((BATCH, _SLOT), jnp.float32),
        jax.ShapeDtypeStruct((16, _SLOT), jnp.float32),
    ),
    scratch_types=[
        pltpu.VMEM((2048,), jnp.int32),
        pltpu.VMEM((_BKT_CAP,), jnp.int32),
        pltpu.VMEM((256,), jnp.int32),
        pltpu.VMEM((256,), jnp.int32),
        pltpu.VMEM((2, EMB_DIM, _GROUP), jnp.float32),
        pltpu.VMEM((64, _SLOT), jnp.float32),
        pltpu.VMEM((EMB_DIM, 64), jnp.float32),
        pltpu.SemaphoreType.DMA,
        pltpu.SemaphoreType.DMA,
    ],
    compiler_params=pltpu.CompilerParams(needs_layout_passes=False),
)
def _gather_kernel(ids_hbm, tab_t_hbm, tail_hbm, out_hbm, dummy_hbm,
                   win_v, bkt_v, offs_v, curs_v, buf_v, ring_v,
                   tail_v, sem_g, sem_r):
    core = lax.axis_index("c")
    tec = lax.axis_index("s")
    nlg = (_GPC - tec + 15) // 16
    is_tail_tec = (core == 1) & (tec == 15)

    iota16 = lax.broadcasted_iota(jnp.int32, (16,), 0)
    zeros16 = jnp.zeros((16,), jnp.int32)

    def scal(ref, i):
        v = plsc.load_gather(ref, [jnp.full((16,), i, jnp.int32)])
        return v[0]

    def masks(c, j):
        idv = win_v[pl.ds(j * 16, 16)]
        bv = jnp.full((16,), c * 2048 + j * 16, jnp.int32) + iota16
        gl = (idv >> 8) - core * _GPC
        own_main = (gl >= 0) & (gl < _GPC) & ((gl & 15) == tec)
        own_tail = (gl == _GPC) & is_tail_tec
        m = own_main | own_tail
        lg = lax.max(lax.min(gl >> 4, jnp.full((16,), 255, jnp.int32)),
                     zeros16)
        return idv, bv, m, lg

    def scan_ids(body16):
        def win_body(c, carry):
            pltpu.sync_copy(ids_hbm.at[pl.ds(c * 2048, 2048)], win_v)

            def vec_body(j, carry2):
                body16(c, j)
                return carry2

            return lax.fori_loop(0, 128, vec_body, carry)

        lax.fori_loop(0, BATCH // 2048, win_body, 0)

    for k in range(16):
        curs_v[pl.ds(16 * k, 16)] = zeros16

    def count16(c, j):
        _, _, m, lg = masks(c, j)
        ordv, lastm = plsc.scan_count(lg, m)
        cur = plsc.load_gather(curs_v, [lg])
        plsc.store_scatter(curs_v, [lg], cur + ordv, mask=m & lastm)

    scan_ids(count16)

    running = jnp.int32(0)
    for k in range(16):
        c = curs_v[pl.ds(16 * k, 16)]
        p = (c + 15) & ~15
        s = plsc.cumsum(p)
        offs_v[pl.ds(16 * k, 16)] = s - p + running
        running = running + s[15]
    for k in range(16):
        curs_v[pl.ds(16 * k, 16)] = offs_v[pl.ds(16 * k, 16)]

    def place16(c, j):
        idv, bv, m, lg = masks(c, j)
        ordv, lastm = plsc.scan_count(lg, m)
        cur = plsc.load_gather(curs_v, [lg])
        slot = lax.min(cur + ordv - 1, jnp.full((16,), _BKT_CAP - 1, jnp.int32))
        slot = lax.max(slot, zeros16)
        val = ((idv & (_GROUP - 1)) << 15) | bv
        plsc.store_scatter(bkt_v, [slot], val, mask=m)
        plsc.store_scatter(curs_v, [lg], cur + ordv, mask=m & lastm)

    scan_ids(place16)

    def fetch(lg, p):
        n0 = (core * _GPC + lg * 16 + tec) * _GROUP
        n0 = pl.multiple_of(n0, 128)
        pltpu.async_copy(tab_t_hbm.at[:, pl.ds(n0, _GROUP)], buf_v.at[p], sem_g)

    @pl.when(nlg > 0)
    def _():
        fetch(0, 0)

    rows16 = [jnp.full((16,), 16 * d, jnp.int32) + iota16 for d in range(4)]

    def drain_one():
        pltpu.make_async_copy(dummy_hbm.at[0], ring_v.at[0], sem_r).wait()

    def chunk_work(c, issued, drained, src_ref, lane_cap, nreal):
        val = bkt_v[pl.ds(c * 16, 16)]
        bv = val & 32767
        lanes_v = lax.min(val >> 15, jnp.full((16,), lane_cap, jnp.int32))

        dr = lax.max(issued - drained - jnp.int32(48), jnp.int32(0))

        def dbody(i, cc):
            drain_one()
            return cc

        lax.fori_loop(0, dr, dbody, 0)
        drained = drained + dr

        for l in range(16):
            @pl.when(l < nreal)
            def _():
                lane = jnp.full((16,), lanes_v[l], jnp.int32)
                slot = (issued + l) & 63
                for d in range(4):
                    vals = plsc.load_gather(src_ref, [rows16[d], lane])
                    ring_v[slot, pl.ds(16 * d, 16)] = vals
                pltpu.async_copy(ring_v.at[slot], out_hbm.at[bv[l]], sem_r)
        return issued + nreal, drained

    def group_body(lg, carry):
        issued, drained = carry
        p = lg % 2
        pltpu.make_async_copy(
            tab_t_hbm.at[:, pl.ds(0, _GROUP)], buf_v.at[p], sem_g
        ).wait()

        @pl.when(lg + 1 < nlg)
        def _():
            fetch(lg + 1, 1 - p)

        off = scal(offs_v, lg)
        cstart = off // 16
        real = scal(curs_v, lg) - off
        nck = (real + 15) // 16

        def inner(k, cc):
            iss, drn = cc
            nreal = lax.min(real - 16 * k, jnp.int32(16))
            return chunk_work(cstart + k, iss, drn, buf_v.at[p],
                              _GROUP - 1, nreal)

        return lax.fori_loop(0, nck, inner, (issued, drained))

    issued, drained = lax.fori_loop(
        0, nlg, group_body, (jnp.int32(0), jnp.int32(0)))

    def tail_seg(cc):
        pltpu.sync_copy(tail_hbm, tail_v)
        off = scal(offs_v, _GPC >> 4)
        cstart = off // 16
        real = scal(curs_v, _GPC >> 4) - off
        nck = (real + 15) // 16

        def inner(k, cc2):
            iss, drn = cc2
            nreal = lax.min(real - 16 * k, jnp.int32(16))
            return chunk_work(cstart + k, iss, drn, tail_v, 63, nreal)

        return lax.fori_loop(0, nck, inner, cc)

    issued, drained = lax.cond(
        is_tail_tec, tail_seg, lambda cc: cc, (issued, drained))

    def final_drain(i, cc):
        drain_one()
        return cc

    lax.fori_loop(0, issued - drained, final_drain, 0)


def kernel(ids, table):
    ids32 = ids.astype(jnp.int32)
    tab_t = table.T
    tail = table[_MAIN_END:].T
    out, _ = _gather_kernel(ids32, tab_t, tail)
    return out[:, :EMB_DIM]


# speedup vs baseline: 12.2575x; 1.3981x over previous
import functools

import jax
import jax.numpy as jnp
from jax import lax
from jax.experimental import pallas as pl
from jax.experimental.pallas import tpu as pltpu
from jax.experimental.pallas import tpu_sc as plsc

BATCH = 16384
EMB_DIM = 64
NUM_NODES = 1_000_000

_info = plsc.get_sparse_core_info()
_NC, _NS = _info.num_cores, _info.num_subcores

_MAIN_END = 999936
_GROUP = 256
_GPC = (_MAIN_END // _GROUP) // 2
_MAXLG = (_GPC + 15) // 16
_BKT_CAP = BATCH + _MAXLG * 16 + 16
_SLOT = 128


@functools.partial(
    pl.kernel,
    mesh=plsc.VectorSubcoreMesh(core_axis_name="c", subcore_axis_name="s"),
    out_type=(
        jax.ShapeDtypeStruct((BATCH, _SLOT), jnp.float32),
        jax.ShapeDtypeStruct((16, _SLOT), jnp.float32),
    ),
    scratch_types=[
        pltpu.VMEM((2048,), jnp.int32),
        pltpu.VMEM((_BKT_CAP,), jnp.int32),
        pltpu.VMEM((256,), jnp.int32),
        pltpu.VMEM((256,), jnp.int32),
        pltpu.VMEM((3, EMB_DIM, _GROUP), jnp.float32),
        pltpu.VMEM((64, _SLOT), jnp.float32),
        pltpu.VMEM((EMB_DIM, 64), jnp.float32),
        pltpu.SemaphoreType.DMA,
        pltpu.SemaphoreType.DMA,
    ],
    compiler_params=pltpu.CompilerParams(needs_layout_passes=False),
)
def _gather_kernel(ids_hbm, tab_t_hbm, tail_hbm, out_hbm, dummy_hbm,
                   win_v, bkt_v, offs_v, curs_v, buf_v, ring_v,
                   tail_v, sem_g, sem_r):
    core = lax.axis_index("c")
    tec = lax.axis_index("s")
    nlg = (_GPC - tec + 15) // 16
    is_tail_tec = (core == 1) & (tec == 15)

    iota16 = lax.broadcasted_iota(jnp.int32, (16,), 0)
    zeros16 = jnp.zeros((16,), jnp.int32)

    def scal(ref, i):
        v = plsc.load_gather(ref, [jnp.full((16,), i, jnp.int32)])
        return v[0]

    def masks(c, j):
        idv = win_v[pl.ds(j * 16, 16)]
        bv = jnp.full((16,), c * 2048 + j * 16, jnp.int32) + iota16
        gl = (idv >> 8) - core * _GPC
        own_main = (gl >= 0) & (gl < _GPC) & ((gl & 15) == tec)
        own_tail = (gl == _GPC) & is_tail_tec
        m = own_main | own_tail
        lg = lax.max(lax.min(gl >> 4, jnp.full((16,), 255, jnp.int32)),
                     zeros16)
        return idv, bv, m, lg

    def scan_ids(body16):
        def win_body(c, carry):
            pltpu.sync_copy(ids_hbm.at[pl.ds(c * 2048, 2048)], win_v)

            def vec_body(j, carry2):
                body16(c, j)
                return carry2

            return lax.fori_loop(0, 128, vec_body, carry)

        lax.fori_loop(0, BATCH // 2048, win_body, 0)

    for k in range(16):
        curs_v[pl.ds(16 * k, 16)] = zeros16

    def count16(c, j):
        _, _, m, lg = masks(c, j)
        ordv, lastm = plsc.scan_count(lg, m)
        cur = plsc.load_gather(curs_v, [lg])
        plsc.store_scatter(curs_v, [lg], cur + ordv, mask=m & lastm)

    scan_ids(count16)

    running = jnp.int32(0)
    for k in range(16):
        c = curs_v[pl.ds(16 * k, 16)]
        p = (c + 15) & ~15
        s = plsc.cumsum(p)
        offs_v[pl.ds(16 * k, 16)] = s - p + running
        running = running + s[15]
    for k in range(16):
        curs_v[pl.ds(16 * k, 16)] = offs_v[pl.ds(16 * k, 16)]

    def place16(c, j):
        idv, bv, m, lg = masks(c, j)
        ordv, lastm = plsc.scan_count(lg, m)
        cur = plsc.load_gather(curs_v, [lg])
        slot = lax.min(cur + ordv - 1, jnp.full((16,), _BKT_CAP - 1, jnp.int32))
        slot = lax.max(slot, zeros16)
        val = ((idv & (_GROUP - 1)) << 15) | bv
        plsc.store_scatter(bkt_v, [slot], val, mask=m)
        plsc.store_scatter(curs_v, [lg], cur + ordv, mask=m & lastm)

    scan_ids(place16)

    def fetch(lg, p):
        n0 = (core * _GPC + lg * 16 + tec) * _GROUP
        n0 = pl.multiple_of(n0, 128)
        pltpu.async_copy(tab_t_hbm.at[:, pl.ds(n0, _GROUP)], buf_v.at[p], sem_g)

    @pl.when(nlg > 0)
    def _():
        fetch(0, 0)

    @pl.when(nlg > 1)
    def _():
        fetch(1, 1)

    rows16 = [jnp.full((16,), 16 * d, jnp.int32) + iota16 for d in range(4)]

    def drain_one():
        pltpu.make_async_copy(dummy_hbm.at[0], ring_v.at[0], sem_r).wait()

    def chunk_work(c, issued, drained, src_ref, lane_cap, nreal):
        val = bkt_v[pl.ds(c * 16, 16)]
        bv = val & 32767
        lanes_v = lax.min(val >> 15, jnp.full((16,), lane_cap, jnp.int32))

        dr = lax.max(issued - drained - jnp.int32(48), jnp.int32(0))

        def dbody(i, cc):
            drain_one()
            return cc

        lax.fori_loop(0, dr, dbody, 0)
        drained = drained + dr

        for l in range(16):
            @pl.when(l < nreal)
            def _():
                lane = jnp.full((16,), lanes_v[l], jnp.int32)
                slot = (issued + l) & 63
                for d in range(4):
                    vals = plsc.load_gather(src_ref, [rows16[d], lane])
                    ring_v[slot, pl.ds(16 * d, 16)] = vals
                pltpu.async_copy(ring_v.at[slot], out_hbm.at[bv[l]], sem_r)
        return issued + nreal, drained

    def group_body(lg, carry):
        issued, drained = carry
        p = lg % 3
        pltpu.make_async_copy(
            tab_t_hbm.at[:, pl.ds(0, _GROUP)], buf_v.at[p], sem_g
        ).wait()

        @pl.when(lg + 2 < nlg)
        def _():
            fetch(lg + 2, (lg + 2) % 3)

        off = scal(offs_v, lg)
        cstart = off // 16
        real = scal(curs_v, lg) - off
        nck = (real + 15) // 16

        def inner(k, cc):
            iss, drn = cc
            nreal = lax.min(real - 16 * k, jnp.int32(16))
            return chunk_work(cstart + k, iss, drn, buf_v.at[p],
                              _GROUP - 1, nreal)

        return lax.fori_loop(0, nck, inner, (issued, drained))

    issued, drained = lax.fori_loop(
        0, nlg, group_body, (jnp.int32(0), jnp.int32(0)))

    def tail_seg(cc):
        pltpu.sync_copy(tail_hbm, tail_v)
        off = scal(offs_v, _GPC >> 4)
        cstart = off // 16
        real = scal(curs_v, _GPC >> 4) - off
        nck = (real + 15) // 16

        def inner(k, cc2):
            iss, drn = cc2
            nreal = lax.min(real - 16 * k, jnp.int32(16))
            return chunk_work(cstart + k, iss, drn, tail_v, 63, nreal)

        return lax.fori_loop(0, nck, inner, cc)

    issued, drained = lax.cond(
        is_tail_tec, tail_seg, lambda cc: cc, (issued, drained))

    def final_drain(i, cc):
        drain_one()
        return cc

    lax.fori_loop(0, issued - drained, final_drain, 0)


def kernel(ids, table):
    ids32 = ids.astype(jnp.int32)
    tab_t = table.T
    tail = table[_MAIN_END:].T
    out, _ = _gather_kernel(ids32, tab_t, tail)
    return out[:, :EMB_DIM]


# speedup vs baseline: 13.0277x; 1.0628x over previous
import functools

import jax
import jax.numpy as jnp
from jax import lax
from jax.experimental import pallas as pl
from jax.experimental.pallas import tpu as pltpu
from jax.experimental.pallas import tpu_sc as plsc

BATCH = 16384
EMB_DIM = 64
NUM_NODES = 1_000_000

_info = plsc.get_sparse_core_info()
_NC, _NS = _info.num_cores, _info.num_subcores

_MAIN_END = 999936
_GROUP = 256
_GPC = (_MAIN_END // _GROUP) // 2
_MAXLG = (_GPC + 15) // 16
_BKT_CAP = BATCH + _MAXLG * 16 + 16
_SLOT = 128


@functools.partial(
    pl.kernel,
    mesh=plsc.VectorSubcoreMesh(core_axis_name="c", subcore_axis_name="s"),
    out_type=(
        jax.ShapeDtypeStruct((BATCH, _SLOT), jnp.float32),
        jax.ShapeDtypeStruct((16, _SLOT), jnp.float32),
    ),
    scratch_types=[
        pltpu.VMEM((2048,), jnp.int32),
        pltpu.VMEM((_BKT_CAP,), jnp.int32),
        pltpu.VMEM((256,), jnp.int32),
        pltpu.VMEM((256,), jnp.int32),
        pltpu.VMEM((4, EMB_DIM, _GROUP), jnp.float32),
        pltpu.VMEM((64, _SLOT), jnp.float32),
        pltpu.VMEM((EMB_DIM, 64), jnp.float32),
        pltpu.SemaphoreType.DMA,
        pltpu.SemaphoreType.DMA,
    ],
    compiler_params=pltpu.CompilerParams(needs_layout_passes=False),
)
def _gather_kernel(ids_hbm, tab_t_hbm, tail_hbm, out_hbm, dummy_hbm,
                   win_v, bkt_v, offs_v, curs_v, buf_v, ring_v,
                   tail_v, sem_g, sem_r):
    core = lax.axis_index("c")
    tec = lax.axis_index("s")
    nlg = (_GPC - tec + 15) // 16
    is_tail_tec = (core == 1) & (tec == 15)

    iota16 = lax.broadcasted_iota(jnp.int32, (16,), 0)
    zeros16 = jnp.zeros((16,), jnp.int32)

    def scal(ref, i):
        v = plsc.load_gather(ref, [jnp.full((16,), i, jnp.int32)])
        return v[0]

    def masks(c, j):
        idv = win_v[pl.ds(j * 16, 16)]
        bv = jnp.full((16,), c * 2048 + j * 16, jnp.int32) + iota16
        gl = (idv >> 8) - core * _GPC
        own_main = (gl >= 0) & (gl < _GPC) & ((gl & 15) == tec)
        own_tail = (gl == _GPC) & is_tail_tec
        m = own_main | own_tail
        lg = lax.max(lax.min(gl >> 4, jnp.full((16,), 255, jnp.int32)),
                     zeros16)
        return idv, bv, m, lg

    def scan_ids(body16):
        def win_body(c, carry):
            pltpu.sync_copy(ids_hbm.at[pl.ds(c * 2048, 2048)], win_v)

            def vec_body(j, carry2):
                body16(c, j)
                return carry2

            return lax.fori_loop(0, 128, vec_body, carry)

        lax.fori_loop(0, BATCH // 2048, win_body, 0)

    for k in range(16):
        curs_v[pl.ds(16 * k, 16)] = zeros16

    def count16(c, j):
        _, _, m, lg = masks(c, j)
        ordv, lastm = plsc.scan_count(lg, m)
        cur = plsc.load_gather(curs_v, [lg])
        plsc.store_scatter(curs_v, [lg], cur + ordv, mask=m & lastm)

    scan_ids(count16)

    running = jnp.int32(0)
    for k in range(16):
        c = curs_v[pl.ds(16 * k, 16)]
        p = (c + 15) & ~15
        s = plsc.cumsum(p)
        offs_v[pl.ds(16 * k, 16)] = s - p + running
        running = running + s[15]
    for k in range(16):
        curs_v[pl.ds(16 * k, 16)] = offs_v[pl.ds(16 * k, 16)]

    def place16(c, j):
        idv, bv, m, lg = masks(c, j)
        ordv, lastm = plsc.scan_count(lg, m)
        cur = plsc.load_gather(curs_v, [lg])
        slot = lax.min(cur + ordv - 1, jnp.full((16,), _BKT_CAP - 1, jnp.int32))
        slot = lax.max(slot, zeros16)
        val = ((idv & (_GROUP - 1)) << 15) | bv
        plsc.store_scatter(bkt_v, [slot], val, mask=m)
        plsc.store_scatter(curs_v, [lg], cur + ordv, mask=m & lastm)

    scan_ids(place16)

    def fetch(lg, p):
        n0 = (core * _GPC + lg * 16 + tec) * _GROUP
        n0 = pl.multiple_of(n0, 128)
        pltpu.async_copy(tab_t_hbm.at[:, pl.ds(n0, _GROUP)], buf_v.at[p], sem_g)

    @pl.when(nlg > 0)
    def _():
        fetch(0, 0)

    @pl.when(nlg > 1)
    def _():
        fetch(1, 1)

    @pl.when(nlg > 2)
    def _():
        fetch(2, 2)

    rows16 = [jnp.full((16,), 16 * d, jnp.int32) + iota16 for d in range(4)]

    def drain_one():
        pltpu.make_async_copy(dummy_hbm.at[0], ring_v.at[0], sem_r).wait()

    def chunk_work(c, issued, drained, src_ref, lane_cap, nreal):
        val = bkt_v[pl.ds(c * 16, 16)]
        bv = val & 32767
        lanes_v = lax.min(val >> 15, jnp.full((16,), lane_cap, jnp.int32))

        dr = lax.max(issued - drained - jnp.int32(48), jnp.int32(0))

        def dbody(i, cc):
            drain_one()
            return cc

        lax.fori_loop(0, dr, dbody, 0)
        drained = drained + dr

        for l in range(16):
            @pl.when(l < nreal)
            def _():
                lane = jnp.full((16,), lanes_v[l], jnp.int32)
                slot = (issued + l) & 63
                for d in range(4):
                    vals = plsc.load_gather(src_ref, [rows16[d], lane])
                    ring_v[slot, pl.ds(16 * d, 16)] = vals
                pltpu.async_copy(ring_v.at[slot], out_hbm.at[bv[l]], sem_r)
        return issued + nreal, drained

    def group_body(lg, carry):
        issued, drained = carry
        p = lg % 4
        pltpu.make_async_copy(
            tab_t_hbm.at[:, pl.ds(0, _GROUP)], buf_v.at[p], sem_g
        ).wait()

        @pl.when(lg + 3 < nlg)
        def _():
            fetch(lg + 3, (lg + 3) % 4)

        off = scal(offs_v, lg)
        cstart = off // 16
        real = scal(curs_v, lg) - off
        nck = (real + 15) // 16

        def inner(k, cc):
            iss, drn = cc
            nreal = lax.min(real - 16 * k, jnp.int32(16))
            return chunk_work(cstart + k, iss, drn, buf_v.at[p],
                              _GROUP - 1, nreal)

        return lax.fori_loop(0, nck, inner, (issued, drained))

    issued, drained = lax.fori_loop(
        0, nlg, group_body, (jnp.int32(0), jnp.int32(0)))

    def tail_seg(cc):
        pltpu.sync_copy(tail_hbm, tail_v)
        off = scal(offs_v, _GPC >> 4)
        cstart = off // 16
        real = scal(curs_v, _GPC >> 4) - off
        nck = (real + 15) // 16

        def inner(k, cc2):
            iss, drn = cc2
            nreal = lax.min(real - 16 * k, jnp.int32(16))
            return chunk_work(cstart + k, iss, drn, tail_v, 63, nreal)

        return lax.fori_loop(0, nck, inner, cc)

    issued, drained = lax.cond(
        is_tail_tec, tail_seg, lambda cc: cc, (issued, drained))

    def final_drain(i, cc):
        drain_one()
        return cc

    lax.fori_loop(0, issued - drained, final_drain, 0)


def kernel(ids, table):
    ids32 = ids.astype(jnp.int32)
    tab_t = table.T
    tail = table[_MAIN_END:].T
    out, _ = _gather_kernel(ids32, tab_t, tail)
    return out[:, :EMB_DIM]
